# Initial kernel scaffold; baseline (speedup 1.0000x reference)
import jax
import jax.numpy as jnp
from jax.experimental import pallas as pl


def kernel(hidden_states, gate_w, w13, w2):
    raise NotImplementedError("write your pallas kernel here")


# speedup vs baseline: 2.1546x; 2.1546x over previous
import functools

import jax
import jax.numpy as jnp
from jax.experimental import pallas as pl
from jax.experimental.pallas import tpu as pltpu

_H, _I, _E = 1024, 512, 8


def _moe_body(x_ref, gate_ref, w13_ref, w2_ref, out_ref, logits_ref, comb_ref):
    e = pl.program_id(0)
    T = x_ref.shape[0]

    @pl.when(e == 0)
    def _router():
        x = x_ref[...]
        logits = jax.lax.dot_general(
            x, gate_ref[...], (((1,), (1,)), ((), ())),
            preferred_element_type=jnp.float32)
        logits_ref[...] = logits
        m = jnp.max(logits, axis=-1, keepdims=True)
        ex = jnp.exp(logits - m)
        p = ex / jnp.sum(ex, axis=-1, keepdims=True)
        ii = jax.lax.broadcasted_iota(jnp.int32, (T, _E), 1)
        m1 = jnp.max(p, axis=-1, keepdims=True)
        idx1 = jnp.min(jnp.where(p == m1, ii, _E), axis=-1, keepdims=True)
        oh1 = ii == idx1
        p2 = jnp.where(oh1, -1.0, p)
        m2 = jnp.max(p2, axis=-1, keepdims=True)
        idx2 = jnp.min(jnp.where(p2 == m2, ii, _E), axis=-1, keepdims=True)
        oh2 = ii == idx2
        s = m1 + m2
        comb_ref[...] = (jnp.where(oh1, m1, 0.0) + jnp.where(oh2, m2, 0.0)) / s

    x = x_ref[...]
    h = jax.lax.dot_general(
        x, w13_ref[0], (((1,), (1,)), ((), ())),
        preferred_element_type=jnp.float32)
    g = h[:, :_I]
    u = h[:, _I:]
    act = (g / (1.0 + jnp.exp(-g))) * u
    y = jax.lax.dot_general(
        act, w2_ref[0], (((1,), (1,)), ((), ())),
        preferred_element_type=jnp.float32)
    ii_e = jax.lax.broadcasted_iota(jnp.int32, (T, _E), 1)
    cw = jnp.sum(jnp.where(ii_e == e, comb_ref[...], 0.0), axis=-1, keepdims=True)
    contrib = cw * y

    @pl.when(e == 0)
    def _init():
        out_ref[...] = contrib

    @pl.when(e > 0)
    def _acc():
        out_ref[...] += contrib


def kernel(hidden_states, gate_w, w13, w2):
    orig = hidden_states.shape
    x = hidden_states.reshape(-1, orig[-1])
    T = x.shape[0]
    out, logits = pl.pallas_call(
        _moe_body,
        grid=(_E,),
        in_specs=[
            pl.BlockSpec((T, _H), lambda e: (0, 0)),
            pl.BlockSpec((_E, _H), lambda e: (0, 0)),
            pl.BlockSpec((1, 2 * _I, _H), lambda e: (e, 0, 0)),
            pl.BlockSpec((1, _H, _I), lambda e: (e, 0, 0)),
        ],
        out_specs=[
            pl.BlockSpec((T, _H), lambda e: (0, 0)),
            pl.BlockSpec((T, _E), lambda e: (0, 0)),
        ],
        out_shape=[
            jax.ShapeDtypeStruct((T, _H), jnp.float32),
            jax.ShapeDtypeStruct((T, _E), jnp.float32),
        ],
        scratch_shapes=[pltpu.VMEM((T, _E), jnp.float32)],
        compiler_params=pltpu.CompilerParams(
            dimension_semantics=("arbitrary",)),
    )(x, gate_w, w13, w2)
    return out.reshape(orig), logits


# speedup vs baseline: 2.1631x; 1.0039x over previous
import functools

import jax
import jax.numpy as jnp
from jax.experimental import pallas as pl
from jax.experimental.pallas import tpu as pltpu

_H, _I, _E = 1024, 512, 8


def _moe_body(x_ref, gate_ref, w13_ref, w2_ref, out_ref, logits_ref,
              comb_ref, xb_ref):
    e = pl.program_id(0)
    T = x_ref.shape[0]

    @pl.when(e == 0)
    def _router():
        x = x_ref[...]
        logits = jax.lax.dot_general(
            x, gate_ref[...], (((1,), (1,)), ((), ())),
            preferred_element_type=jnp.float32)
        logits_ref[...] = logits
        m = jnp.max(logits, axis=-1, keepdims=True)
        ex = jnp.exp(logits - m)
        p = ex / jnp.sum(ex, axis=-1, keepdims=True)
        ii = jax.lax.broadcasted_iota(jnp.int32, (T, _E), 1)
        m1 = jnp.max(p, axis=-1, keepdims=True)
        idx1 = jnp.min(jnp.where(p == m1, ii, _E), axis=-1, keepdims=True)
        oh1 = ii == idx1
        p2 = jnp.where(oh1, -1.0, p)
        m2 = jnp.max(p2, axis=-1, keepdims=True)
        idx2 = jnp.min(jnp.where(p2 == m2, ii, _E), axis=-1, keepdims=True)
        oh2 = ii == idx2
        s = m1 + m2
        comb_ref[...] = (jnp.where(oh1, m1, 0.0) + jnp.where(oh2, m2, 0.0)) / s
        xb_ref[...] = x.astype(jnp.bfloat16)

    xb = xb_ref[...]
    h = jax.lax.dot_general(
        xb, w13_ref[0].astype(jnp.bfloat16), (((1,), (1,)), ((), ())),
        preferred_element_type=jnp.float32)
    g = h[:, :_I]
    u = h[:, _I:]
    act = (g / (1.0 + jnp.exp(-g))) * u
    y = jax.lax.dot_general(
        act.astype(jnp.bfloat16), w2_ref[0].astype(jnp.bfloat16),
        (((1,), (1,)), ((), ())),
        preferred_element_type=jnp.float32)
    ii_e = jax.lax.broadcasted_iota(jnp.int32, (T, _E), 1)
    cw = jnp.sum(jnp.where(ii_e == e, comb_ref[...], 0.0), axis=-1, keepdims=True)
    contrib = cw * y

    @pl.when(e == 0)
    def _init():
        out_ref[...] = contrib

    @pl.when(e > 0)
    def _acc():
        out_ref[...] += contrib


def kernel(hidden_states, gate_w, w13, w2):
    orig = hidden_states.shape
    x = hidden_states.reshape(-1, orig[-1])
    T = x.shape[0]
    out, logits = pl.pallas_call(
        _moe_body,
        grid=(_E,),
        in_specs=[
            pl.BlockSpec((T, _H), lambda e: (0, 0)),
            pl.BlockSpec((_E, _H), lambda e: (0, 0)),
            pl.BlockSpec((1, 2 * _I, _H), lambda e: (e, 0, 0)),
            pl.BlockSpec((1, _H, _I), lambda e: (e, 0, 0)),
        ],
        out_specs=[
            pl.BlockSpec((T, _H), lambda e: (0, 0)),
            pl.BlockSpec((T, _E), lambda e: (0, 0)),
        ],
        out_shape=[
            jax.ShapeDtypeStruct((T, _H), jnp.float32),
            jax.ShapeDtypeStruct((T, _E), jnp.float32),
        ],
        scratch_shapes=[pltpu.VMEM((T, _E), jnp.float32),
                        pltpu.VMEM((T, _H), jnp.bfloat16)],
        compiler_params=pltpu.CompilerParams(
            dimension_semantics=("arbitrary",)),
    )(x, gate_w, w13, w2)
    return out.reshape(orig), logits
